# baseline (speedup 1.0000x reference)
import functools

import jax
import jax.numpy as jnp
from jax import lax
from jax.experimental import pallas as pl
from jax.experimental.pallas import tpu as pltpu
from jax.experimental.pallas import tpu_sc as plsc

EMBED = 64

_NC = 2
_NS = 16
_NW = _NC * _NS

_IDX_BATCH = 128
_BATCHES_PER_CHUNK = 8
_CHUNK = _IDX_BATCH * _BATCHES_PER_CHUNK


def _sc_gather(table, idx2d, n_rows):
    rows_per_w = n_rows // _NW
    chunks_per_w = rows_per_w // _CHUNK
    mesh = plsc.VectorSubcoreMesh(core_axis_name="c", subcore_axis_name="s")

    @functools.partial(
        pl.kernel,
        mesh=mesh,
        out_type=jax.ShapeDtypeStruct((n_rows, EMBED), jnp.float32),
        compiler_params=pltpu.CompilerParams(use_tc_tiling_on_sc=False),
        scratch_types=[
            pltpu.VMEM((_BATCHES_PER_CHUNK, _IDX_BATCH), jnp.int32),
            pltpu.VMEM((_CHUNK, EMBED), jnp.float32),
            pltpu.SemaphoreType.DMA,
        ],
    )
    def gather_kernel(idx_hbm, table_hbm, out_hbm, idx_v, rows_v, sem):
        wid = lax.axis_index("s") * _NC + lax.axis_index("c")
        base = wid * rows_per_w

        def chunk_body(c, carry):
            row0 = pl.multiple_of(base + c * _CHUNK, _CHUNK)
            pltpu.sync_copy(
                idx_hbm.at[
                    pl.ds(
                        pl.multiple_of(row0 // _IDX_BATCH, _BATCHES_PER_CHUNK),
                        _BATCHES_PER_CHUNK,
                    )
                ],
                idx_v,
            )
            handles = []
            for j in range(_BATCHES_PER_CHUNK):
                handles.append(
                    pltpu.async_copy(
                        table_hbm.at[idx_v.at[j]],
                        rows_v.at[pl.ds(j * _IDX_BATCH, _IDX_BATCH)],
                        sem,
                    )
                )
            for h in handles:
                h.wait()
            pltpu.sync_copy(rows_v, out_hbm.at[pl.ds(row0, _CHUNK)])
            return carry

        lax.fori_loop(0, chunks_per_w, chunk_body, 0)

    return gather_kernel(idx2d, table)


def _mlp_body(emb, W1d, b1d, msk, W2t, b2c, out):
    h = jnp.dot(emb[...], W1d[...], preferred_element_type=jnp.float32)
    h = jnp.maximum(h + b1d[...], 0.0)
    msk_ = msk[...]
    m = jnp.dot(h, msk_, preferred_element_type=jnp.float32)
    d = h - m
    v = jnp.dot(d * d, msk_, preferred_element_type=jnp.float32)
    hn = d * lax.rsqrt(v + 1e-5)
    W2t_ = W2t[...]
    b2c_ = b2c[...]
    dn = (((1,), (1,)), ((), ()))
    out[0] = lax.dot_general(W2t_, hn[:, :EMBED], dn,
                             preferred_element_type=jnp.float32) + b2c_
    out[1] = lax.dot_general(W2t_, hn[:, EMBED:], dn,
                             preferred_element_type=jnp.float32) + b2c_


def _tc_mlp(emb128, W1d, b1d, msk, W2t, b2c, n_b, n_l, blk):
    n_half = emb128.shape[0]
    nb = n_b // blk
    nl2 = n_l // 2
    vfull = pl.BlockSpec((1, 2 * EMBED), lambda i: (0, 0))
    wfull = pl.BlockSpec((2 * EMBED, 2 * EMBED), lambda i: (0, 0))
    return pl.pallas_call(
        _mlp_body,
        grid=(nl2 * nb,),
        in_specs=[
            pl.BlockSpec((blk, 2 * EMBED), lambda i: (i, 0)),
            wfull, vfull, wfull,
            pl.BlockSpec((EMBED, EMBED), lambda i: (0, 0)),
            pl.BlockSpec((EMBED, 1), lambda i: (0, 0)),
        ],
        out_specs=pl.BlockSpec(
            (2, EMBED, blk), lambda i: (0, i // nb, i % nb)),
        out_shape=jax.ShapeDtypeStruct((2, nl2 * EMBED, n_b), jnp.float32),
    )(emb128, W1d, b1d, msk, W2t, b2c)


def _dup_diag(W):
    z = jnp.zeros_like(W)
    return jnp.concatenate(
        [jnp.concatenate([W, z], axis=1), jnp.concatenate([z, W], axis=1)],
        axis=0)


def kernel(x, table, W1, b1, gamma, beta, W2, b2):
    B, L = x.shape
    n_rows = B * L
    idx_seq = x.astype(jnp.int32).T.reshape(2, n_rows // 2)
    idx2d = idx_seq.T.reshape(n_rows // _IDX_BATCH, _IDX_BATCH)
    emb = _sc_gather(table, idx2d, n_rows)
    emb128 = emb.reshape(n_rows // 2, 2 * EMBED)
    dup = lambda v: jnp.tile(v.reshape(1, EMBED), (1, 2))
    msk = _dup_diag(jnp.full((EMBED, EMBED), 1.0 / EMBED, jnp.float32))
    out4 = _tc_mlp(
        emb128,
        _dup_diag(W1),
        dup(b1),
        msk,
        (W2 * gamma[:, None]).T,
        (beta @ W2 + b2).reshape(EMBED, 1),
        n_b=B,
        n_l=L,
        blk=4096,
    )
    out3 = out4.reshape(L, EMBED, B)
    return out3.transpose(2, 0, 1)


# speedup vs baseline: 1.0301x; 1.0301x over previous
import functools

import jax
import jax.numpy as jnp
from jax import lax
from jax.experimental import pallas as pl
from jax.experimental.pallas import tpu as pltpu
from jax.experimental.pallas import tpu_sc as plsc

EMBED = 64

_NC = 2
_NS = 16
_NW = _NC * _NS

_IDX_BATCH = 128
_BATCHES_PER_CHUNK = 8
_CHUNK = _IDX_BATCH * _BATCHES_PER_CHUNK


def _sc_gather(table, idx2d, n_rows):
    rows_per_w = n_rows // _NW
    chunks_per_w = rows_per_w // _CHUNK
    mesh = plsc.VectorSubcoreMesh(core_axis_name="c", subcore_axis_name="s")

    @functools.partial(
        pl.kernel,
        mesh=mesh,
        out_type=jax.ShapeDtypeStruct((n_rows, EMBED), jnp.float32),
        compiler_params=pltpu.CompilerParams(use_tc_tiling_on_sc=False),
        scratch_types=[
            pltpu.VMEM((_BATCHES_PER_CHUNK, _IDX_BATCH), jnp.int32),
            pltpu.VMEM((_CHUNK, EMBED), jnp.float32),
            pltpu.SemaphoreType.DMA,
        ],
    )
    def gather_kernel(idx_hbm, table_hbm, out_hbm, idx_v, rows_v, sem):
        wid = lax.axis_index("s") * _NC + lax.axis_index("c")
        base = wid * rows_per_w

        def chunk_body(c, carry):
            row0 = pl.multiple_of(base + c * _CHUNK, _CHUNK)
            pltpu.sync_copy(
                idx_hbm.at[
                    pl.ds(
                        pl.multiple_of(row0 // _IDX_BATCH, _BATCHES_PER_CHUNK),
                        _BATCHES_PER_CHUNK,
                    )
                ],
                idx_v,
            )
            handles = []
            for j in range(_BATCHES_PER_CHUNK):
                handles.append(
                    pltpu.async_copy(
                        table_hbm.at[idx_v.at[j]],
                        rows_v.at[pl.ds(j * _IDX_BATCH, _IDX_BATCH)],
                        sem,
                    )
                )
            for h in handles:
                h.wait()
            pltpu.sync_copy(rows_v, out_hbm.at[pl.ds(row0, _CHUNK)])
            return carry

        lax.fori_loop(0, chunks_per_w, chunk_body, 0)

    return gather_kernel(idx2d, table)


def _mlp_body(emb, W1d, b1d, msk, W2t, b2c, out):
    h = jnp.dot(emb[...], W1d[...], preferred_element_type=jnp.float32)
    h = jnp.maximum(h + b1d[...], 0.0)
    msk_ = msk[...]
    m = jnp.dot(h, msk_, preferred_element_type=jnp.float32)
    d = h - m
    v = jnp.dot(d * d, msk_, preferred_element_type=jnp.float32)
    hn = d * lax.rsqrt(v + 1e-5)
    W2t_ = W2t[...]
    b2c_ = b2c[...]
    dn = (((1,), (1,)), ((), ()))
    out[0] = lax.dot_general(W2t_, hn[:, :EMBED], dn,
                             preferred_element_type=jnp.float32) + b2c_
    out[1] = lax.dot_general(W2t_, hn[:, EMBED:], dn,
                             preferred_element_type=jnp.float32) + b2c_


def _tc_mlp(emb128, W1d, b1d, msk, W2t, b2c, n_b, n_l, blk):
    n_half = emb128.shape[0]
    nb = n_b // blk
    nl2 = n_l // 2
    vfull = pl.BlockSpec((1, 2 * EMBED), lambda i: (0, 0))
    wfull = pl.BlockSpec((2 * EMBED, 2 * EMBED), lambda i: (0, 0))
    return pl.pallas_call(
        _mlp_body,
        grid=(nl2 * nb,),
        in_specs=[
            pl.BlockSpec((blk, 2 * EMBED), lambda i: (i, 0)),
            wfull, vfull, wfull,
            pl.BlockSpec((EMBED, EMBED), lambda i: (0, 0)),
            pl.BlockSpec((EMBED, 1), lambda i: (0, 0)),
        ],
        out_specs=pl.BlockSpec(
            (2, EMBED, blk), lambda i: (0, i // nb, i % nb)),
        out_shape=jax.ShapeDtypeStruct((2, nl2 * EMBED, n_b), jnp.float32),
    )(emb128, W1d, b1d, msk, W2t, b2c)


def _dup_diag(W):
    z = jnp.zeros_like(W)
    return jnp.concatenate(
        [jnp.concatenate([W, z], axis=1), jnp.concatenate([z, W], axis=1)],
        axis=0)


def kernel(x, table, W1, b1, gamma, beta, W2, b2):
    B, L = x.shape
    n_rows = B * L
    idx_seq = x.astype(jnp.int32).T.reshape(2, n_rows // 2)
    idx2d = idx_seq.T.reshape(n_rows // _IDX_BATCH, _IDX_BATCH)
    emb = _sc_gather(table, idx2d, n_rows)
    emb128 = emb.reshape(n_rows // 2, 2 * EMBED)
    dup = lambda v: jnp.tile(v.reshape(1, EMBED), (1, 2))
    msk = _dup_diag(jnp.full((EMBED, EMBED), 1.0 / EMBED, jnp.float32))
    out4 = _tc_mlp(
        emb128,
        _dup_diag(W1),
        dup(b1),
        msk,
        (W2 * gamma[:, None]).T,
        (beta @ W2 + b2).reshape(EMBED, 1),
        n_b=B,
        n_l=L,
        blk=8192,
    )
    out3 = out4.reshape(L, EMBED, B)
    return out3.transpose(2, 0, 1)
